# baseline (speedup 1.0000x reference)
import functools
import jax
import jax.numpy as jnp
from jax import lax
from jax.experimental import pallas as pl
from jax.experimental.pallas import tpu as pltpu, tpu_sc as plsc

ROWS = 8


def _make_sc_kernel(B, S, D, d3):
    info = plsc.get_sparse_core_info()
    nw = info.num_cores * info.num_subcores
    s_per_w = S // nw
    g_rows = 3 * s_per_w
    chunk = ROWS * D
    n_chunks_per_b = s_per_w // ROWS
    n_chunks = B * n_chunks_per_b
    n_vec = chunk // 16
    mesh = plsc.VectorSubcoreMesh(core_axis_name="c", subcore_axis_name="s")

    @functools.partial(
        pl.kernel, mesh=mesh,
        out_type=jax.ShapeDtypeStruct((B, S * D), jnp.float32),
        scratch_types=[
            pltpu.VMEM((g_rows,), jnp.int32),
            pltpu.VMEM((g_rows, d3), jnp.float32),
            pltpu.VMEM((chunk,), jnp.float32),
            pltpu.VMEM((chunk,), jnp.float32),
            pltpu.VMEM((chunk,), jnp.float32),
            pltpu.VMEM((chunk,), jnp.float32),
            pltpu.SemaphoreType.DMA,
            pltpu.SemaphoreType.DMA,
            pltpu.SemaphoreType.DMA,
            pltpu.SemaphoreType.DMA,
            pltpu.SemaphoreType.DMA,
        ],
    )
    def sc_kernel(table_hbm, idx_hbm, x_hbm, out_hbm,
                  idx_v, pos_v, in0, in1, out0, out1,
                  gsem, isem0, isem1, osem0, osem1):
        wid = lax.axis_index("s") * info.num_cores + lax.axis_index("c")
        base = wid * s_per_w

        pltpu.sync_copy(idx_hbm.at[pl.ds(wid * g_rows, g_rows)], idx_v)
        gathers = []
        for j in range(g_rows // 128):
            gathers.append(pltpu.async_copy(
                table_hbm.at[idx_v.at[pl.ds(128 * j, 128)]],
                pos_v.at[pl.ds(128 * j, 128)], gsem))
        for g in gathers:
            g.wait()

        ins = (in0, in1)
        outs = (out0, out1)
        isems = (isem0, isem1)
        osems = (osem0, osem1)

        def x_slice(i):
            b, t = divmod(i, n_chunks_per_b)
            return x_hbm.at[b, pl.ds((base + ROWS * t) * D, chunk)]

        def out_slice(i):
            b, t = divmod(i, n_chunks_per_b)
            return out_hbm.at[b, pl.ds((base + ROWS * t) * D, chunk)]

        def add_chunk(i):
            t = i % n_chunks_per_b
            src, dst = ins[i % 2], outs[i % 2]

            def body(k, _):
                g16 = t * n_vec + k
                row = g16 >> 4
                col = pl.multiple_of((g16 & 15) << 4, 16)
                off = pl.multiple_of(k * 16, 16)
                dst[pl.ds(off, 16)] = (
                    src[pl.ds(off, 16)] + pos_v[row, pl.ds(col, 16)])
                return _

            lax.fori_loop(0, n_vec, body, None)

        in_copies = [None] * n_chunks
        out_copies = [None] * n_chunks
        in_copies[0] = pltpu.async_copy(x_slice(0), ins[0], isems[0])
        in_copies[1] = pltpu.async_copy(x_slice(1), ins[1], isems[1])
        for i in range(n_chunks):
            in_copies[i].wait()
            if i >= 2:
                out_copies[i - 2].wait()
            add_chunk(i)
            out_copies[i] = pltpu.async_copy(
                outs[i % 2], out_slice(i), osems[i % 2])
            if i + 2 < n_chunks:
                in_copies[i + 2] = pltpu.async_copy(
                    x_slice(i + 2), ins[i % 2], isems[i % 2])
        out_copies[n_chunks - 2].wait()
        out_copies[n_chunks - 1].wait()

    return sc_kernel


def kernel(x, src_tgt, emb_x, emb_y, emb_z, src_pos_x, src_pos_y, src_pos_z):
    B, S, D = x.shape
    d3 = emb_x.shape[1]
    nx, ny, nz = emb_x.shape[0], emb_y.shape[0], emb_z.shape[0]

    is_src = (src_tgt != 0)
    sx = jnp.concatenate([jnp.array([nx - 1], jnp.int32), src_pos_x])[:S]
    sy = jnp.concatenate([jnp.array([ny - 1], jnp.int32), src_pos_y])[:S]
    sz = jnp.concatenate([jnp.array([nz - 1], jnp.int32), src_pos_z])[:S]
    px = jnp.where(is_src, src_pos_x, sx)
    py = jnp.where(is_src, src_pos_y, sy) + nx
    pz = jnp.where(is_src, src_pos_z, sz) + nx + ny

    table = jnp.concatenate([emb_x, emb_y, emb_z], axis=0)
    idx = jnp.stack([px, py, pz], axis=1).reshape(-1)

    out = _make_sc_kernel(B, S, D, d3)(table, idx, x.reshape(B, S * D))
    return out.reshape(B, S, D)


# speedup vs baseline: 3.3030x; 3.3030x over previous
import functools
import jax
import jax.numpy as jnp
from jax import lax
from jax.experimental import pallas as pl
from jax.experimental.pallas import tpu as pltpu, tpu_sc as plsc

BS = 512


def _add_body(posx_ref, posyz_ref, x_ref, out_ref):
    d3 = posx_ref.shape[-1]
    out_ref[:, :, :d3] = x_ref[:, :, :d3] + posx_ref[...][None]
    out_ref[:, :, d3:] = x_ref[:, :, d3:] + posyz_ref[...][None]


def _make_sc_gather(S, d3):
    info = plsc.get_sparse_core_info()
    nw = info.num_cores * info.num_subcores
    spw = S // nw
    mesh = plsc.VectorSubcoreMesh(core_axis_name="c", subcore_axis_name="s")

    @functools.partial(
        pl.kernel, mesh=mesh,
        out_type=(
            jax.ShapeDtypeStruct((S, d3), jnp.float32),
            jax.ShapeDtypeStruct((S, 2 * d3), jnp.float32),
        ),
        scratch_types=[
            pltpu.VMEM((2, spw), jnp.int32),
            pltpu.VMEM((spw, d3), jnp.float32),
            pltpu.VMEM((spw, 2 * d3), jnp.float32),
            pltpu.SemaphoreType.DMA,
        ],
    )
    def sc_gather(xtab_hbm, yztab_hbm, idx_hbm, posx_hbm, posyz_hbm,
                  idx_v, rx_v, ryz_v, sem):
        wid = lax.axis_index("s") * info.num_cores + lax.axis_index("c")
        base = wid * spw
        pltpu.sync_copy(idx_hbm.at[:, pl.ds(base, spw)], idx_v)
        cx = pltpu.async_copy(xtab_hbm.at[idx_v.at[0]], rx_v, sem)
        cyz = pltpu.async_copy(yztab_hbm.at[idx_v.at[1]], ryz_v, sem)
        cx.wait()
        cyz.wait()
        pltpu.sync_copy(rx_v, posx_hbm.at[pl.ds(base, spw)])
        pltpu.sync_copy(ryz_v, posyz_hbm.at[pl.ds(base, spw)])

    return sc_gather


def kernel(x, src_tgt, emb_x, emb_y, emb_z, src_pos_x, src_pos_y, src_pos_z):
    B, S, D = x.shape
    d3 = emb_x.shape[1]
    nx, ny, nz = emb_x.shape[0], emb_y.shape[0], emb_z.shape[0]

    is_src = (src_tgt != 0)
    sx = jnp.concatenate([jnp.array([nx - 1], jnp.int32), src_pos_x])[:S]
    sy = jnp.concatenate([jnp.array([ny - 1], jnp.int32), src_pos_y])[:S]
    sz = jnp.concatenate([jnp.array([nz - 1], jnp.int32), src_pos_z])[:S]
    px = jnp.where(is_src, src_pos_x, sx)
    py = jnp.where(is_src, src_pos_y, sy)
    pz = jnp.where(is_src, src_pos_z, sz)

    yztab = jnp.concatenate(
        [jnp.repeat(emb_y, nz, axis=0), jnp.tile(emb_z, (ny, 1))], axis=1)
    idx = jnp.stack([px, py * nz + pz])

    posx, posyz = _make_sc_gather(S, d3)(emb_x, yztab, idx)

    nb = S // BS
    out = pl.pallas_call(
        _add_body,
        grid=(nb,),
        in_specs=[
            pl.BlockSpec((BS, d3), lambda i: (i, 0)),
            pl.BlockSpec((BS, 2 * d3), lambda i: (i, 0)),
            pl.BlockSpec((B, BS, D), lambda i: (0, i, 0)),
        ],
        out_specs=pl.BlockSpec((B, BS, D), lambda i: (0, i, 0)),
        out_shape=jax.ShapeDtypeStruct((B, S, D), jnp.float32),
    )(posx, posyz, x)
    return out


# speedup vs baseline: 3.8903x; 1.1778x over previous
import functools
import jax
import jax.numpy as jnp
from jax import lax
from jax.experimental import pallas as pl
from jax.experimental.pallas import tpu as pltpu, tpu_sc as plsc

BS = 512


def _add_body(idxx_ref, ex_ref, posyz_ref, x_ref, out_ref):
    d3 = ex_ref.shape[-1]
    nrow = ex_ref.shape[0]
    bs = idxx_ref.shape[-1]
    ix = idxx_ref[0, 0]
    iot = lax.broadcasted_iota(jnp.int32, (bs, nrow), 1)
    oh = (iot == ix[:, None]).astype(jnp.float32)
    posx = jnp.dot(oh, ex_ref[...], preferred_element_type=jnp.float32)
    out_ref[:, :, :d3] = x_ref[:, :, :d3] + posx[None]
    out_ref[:, :, d3:] = x_ref[:, :, d3:] + posyz_ref[...][None]


def _make_sc_gather(S, d3):
    info = plsc.get_sparse_core_info()
    nw = info.num_cores * info.num_subcores
    spw = S // nw
    mesh = plsc.VectorSubcoreMesh(core_axis_name="c", subcore_axis_name="s")

    @functools.partial(
        pl.kernel, mesh=mesh,
        out_type=jax.ShapeDtypeStruct((S, 2 * d3), jnp.float32),
        scratch_types=[
            pltpu.VMEM((spw,), jnp.int32),
            pltpu.VMEM((spw, 2 * d3), jnp.float32),
            pltpu.SemaphoreType.DMA,
        ],
    )
    def sc_gather(yztab_hbm, idx_hbm, posyz_hbm, idx_v, ryz_v, sem):
        wid = lax.axis_index("s") * info.num_cores + lax.axis_index("c")
        base = wid * spw
        pltpu.sync_copy(idx_hbm.at[pl.ds(base, spw)], idx_v)
        pltpu.async_copy(yztab_hbm.at[idx_v], ryz_v, sem).wait()
        pltpu.sync_copy(ryz_v, posyz_hbm.at[pl.ds(base, spw)])

    return sc_gather


def kernel(x, src_tgt, emb_x, emb_y, emb_z, src_pos_x, src_pos_y, src_pos_z):
    B, S, D = x.shape
    d3 = emb_x.shape[1]
    nx, ny, nz = emb_x.shape[0], emb_y.shape[0], emb_z.shape[0]

    is_src = (src_tgt != 0)
    sx = jnp.concatenate([jnp.array([nx - 1], jnp.int32), src_pos_x])[:S]
    sy = jnp.concatenate([jnp.array([ny - 1], jnp.int32), src_pos_y])[:S]
    sz = jnp.concatenate([jnp.array([nz - 1], jnp.int32), src_pos_z])[:S]
    px = jnp.where(is_src, src_pos_x, sx)
    py = jnp.where(is_src, src_pos_y, sy)
    pz = jnp.where(is_src, src_pos_z, sz)

    yztab = jnp.concatenate(
        [jnp.repeat(emb_y, nz, axis=0), jnp.tile(emb_z, (ny, 1))], axis=1)
    posyz = _make_sc_gather(S, d3)(yztab, py * nz + pz)

    nxp = (nx + 7) // 8 * 8
    ex = jnp.zeros((nxp, d3), jnp.float32).at[:nx].set(emb_x)

    nb = S // BS
    idxx = px.reshape(nb, 1, BS)
    out = pl.pallas_call(
        _add_body,
        grid=(nb,),
        in_specs=[
            pl.BlockSpec((1, 1, BS), lambda i: (i, 0, 0)),
            pl.BlockSpec((nxp, d3), lambda i: (0, 0)),
            pl.BlockSpec((BS, 2 * d3), lambda i: (i, 0)),
            pl.BlockSpec((B, BS, D), lambda i: (0, i, 0)),
        ],
        out_specs=pl.BlockSpec((B, BS, D), lambda i: (0, i, 0)),
        out_shape=jax.ShapeDtypeStruct((B, S, D), jnp.float32),
    )(idxx, ex, posyz, x)
    return out
